# Initial kernel scaffold; baseline (speedup 1.0000x reference)
import jax
import jax.numpy as jnp
from jax.experimental import pallas as pl


def kernel(x, edge_index, edge_attr, pos, batch, W1, R1, g1, b1, W2, R2, g2, b2, W3, R3, g3, b3, Wr, Rr, br, Wc, Rc, bc, Wo, Ro, bo):
    raise NotImplementedError("write your pallas kernel here")


# speedup vs baseline: 1.1640x; 1.1640x over previous
import functools

import jax
import jax.numpy as jnp
from jax import lax
from jax.experimental import pallas as pl
from jax.experimental.pallas import tpu as pltpu
from jax.experimental.pallas import tpu_sc as plsc

N = 10000
E = 160000
C = 128
KT = 25
KS = 5
KTH = 32
B = 16
GH, GW = 12, 16
NCELL = B * GH * GW

NC, NS = 2, 16
NW = NC * NS
EW = E // NW
CH = 40
NCHUNK = EW // CH
NP = 10240
RW = NP // NS
RP = 128
CELL_W = NCELL // NW

def _get_mesh():
    return plsc.VectorSubcoreMesh(core_axis_name="c", subcore_axis_name="s",
                                  num_cores=NC, num_subcores=NS)


def _mm_body(x_ref, w_ref, o_ref):
    o_ref[...] = jnp.dot(x_ref[...], w_ref[...],
                         preferred_element_type=jnp.float32)


def _mm(x, w):
    n, k = x.shape
    m = w.shape[1]
    bn = 2000
    return pl.pallas_call(
        _mm_body,
        grid=(n // bn, m // 128),
        in_specs=[pl.BlockSpec((bn, k), lambda i, j: (i, 0)),
                  pl.BlockSpec((k, 128), lambda i, j: (0, j))],
        out_specs=pl.BlockSpec((bn, 128), lambda i, j: (i, j)),
        out_shape=jax.ShapeDtypeStruct((n, m), jnp.float32),
    )(x, w)


def _prep_body(ea_ref, bas_ref, gi_ref):
    v0 = ea_ref[0:1] * (KS - 1.0)
    v1 = ea_ref[1:2] * (KS - 1.0)
    srcf = ea_ref[2:3]
    lo0 = jnp.floor(v0)
    fr0 = v0 - lo0
    lo1 = jnp.floor(v1)
    fr1 = v1 - lo1
    rows_b = []
    rows_gb = []
    rows_gh = []
    for s0 in (0, 1):
        for s1 in (0, 1):
            b0 = fr0 if s0 else 1.0 - fr0
            b1 = fr1 if s1 else 1.0 - fr1
            i0 = jnp.clip(lo0 + s0, 0.0, KS - 1.0)
            i1 = jnp.clip(lo1 + s1, 0.0, KS - 1.0)
            wi = i0 + i1 * KS
            rows_b.append(b0 * b1)
            rows_gb.append(srcf * KT + wi)
            rows_gh.append(srcf * KTH + wi)
    bas_ref[...] = jnp.concatenate(rows_b + rows_b, axis=0)
    gi = jnp.concatenate(rows_gb + rows_gh, axis=0)
    gi_ref[...] = gi.astype(jnp.int32)


def _prep(ea8):
    be = 1280
    return pl.pallas_call(
        _prep_body,
        grid=(E // be,),
        in_specs=[pl.BlockSpec((8, be), lambda i: (0, i))],
        out_specs=[pl.BlockSpec((8, be), lambda i: (0, i)),
                   pl.BlockSpec((8, be), lambda i: (0, i))],
        out_shape=[jax.ShapeDtypeStruct((8, E), jnp.float32),
                   jax.ShapeDtypeStruct((8, E), jnp.int32)],
    )(ea8)


def _edge_body(ocols, xw_ref, gidx_ref, bas_ref, dst_ref, zrow_ref,
               agg_out, idxa_v, idxb_v, bas_v, dst_v, rows_v, msg_v, agg_sp,
               sem):

    cid = lax.axis_index("c")
    sid = lax.axis_index("s")
    wid = cid * NS + sid

    zb_v = rows_v.at[pl.ds(0, RP)]
    pltpu.sync_copy(zrow_ref, zb_v)
    for t in range(RW // RP):
        pltpu.sync_copy(zb_v, agg_sp.at[pl.ds(sid * RW + t * RP, RP)])
    plsc.subcore_barrier()

    base = wid * EW
    nsub = C // 16 if ocols == C else 1

    @pl.loop(0, NCHUNK)
    def _chunk(it):
        e0 = base + it * CH
        pltpu.sync_copy(gidx_ref.at[pl.ds(e0 * 4, 2 * CH)], idxa_v)
        pltpu.sync_copy(gidx_ref.at[pl.ds(e0 * 4 + 2 * CH, 2 * CH)], idxb_v)
        pltpu.sync_copy(bas_ref.at[pl.ds(e0 * 4, 4 * CH)],
                        bas_v.at[pl.ds(0, 4 * CH)])
        pltpu.sync_copy(dst_ref.at[pl.ds(e0, CH)], dst_v)
        cpa = pltpu.async_copy(xw_ref.at[idxa_v],
                               rows_v.at[pl.ds(0, 2 * CH)], sem)
        cpb = pltpu.async_copy(xw_ref.at[idxb_v],
                               rows_v.at[pl.ds(2 * CH, 2 * CH)], sem)
        cpa.wait()
        cpb.wait()

        @pl.loop(0, CH, unroll=2)
        def _edge(j):
            bvec = bas_v[pl.ds(4 * j, 16)]
            for cc in range(nsub):
                acc = jnp.zeros((16,), jnp.float32)
                for s in range(4):
                    acc = acc + bvec[s] * rows_v[4 * j + s, pl.ds(cc * 16, 16)]
                msg_v[j, pl.ds(cc * 16, 16)] = acc

        pltpu.sync_copy(msg_v, agg_sp.at[dst_v], add=True)

    plsc.subcore_barrier()

    zb_v = rows_v.at[pl.ds(0, RP)]
    for t in range(RW // RP):
        r0 = sid * RW + t * RP
        pltpu.sync_copy(agg_sp.at[pl.ds(r0, RP)], zb_v)
        pltpu.sync_copy(zb_v, agg_out.at[cid, pl.ds(r0, RP)])


def _edge_pass(xw_flat, gidx_il, bas_il, dst, ocols):
    scratch = [
        pltpu.VMEM((2 * CH,), jnp.int32),
        pltpu.VMEM((2 * CH,), jnp.int32),
        pltpu.VMEM((4 * CH + 16,), jnp.float32),
        pltpu.VMEM((CH,), jnp.int32),
        pltpu.VMEM((4 * CH, ocols), jnp.float32),
        pltpu.VMEM((CH, ocols), jnp.float32),
        pltpu.VMEM_SHARED((NP, ocols), jnp.float32),
        pltpu.SemaphoreType.DMA,
    ]
    zrow = jnp.zeros((RP, ocols), jnp.float32)
    fn = pl.kernel(
        functools.partial(_edge_body, ocols),
        out_type=jax.ShapeDtypeStruct((NC, NP, ocols), jnp.float32),
        mesh=_get_mesh(),
        scratch_types=scratch,
        compiler_params=pltpu.CompilerParams(
            use_tc_tiling_on_sc=(ocols == C)),
    )
    return fn(xw_flat, gidx_il, bas_il, dst, zrow)


def _cnt_body(dst_ref, z16_ref, one_ref, cnt_out, dst_v, cb_v, ones_v,
              cnt_sp, sem):
    cid = lax.axis_index("c")
    sid = lax.axis_index("s")
    wid = cid * NS + sid
    pltpu.sync_copy(one_ref, ones_v)
    pltpu.sync_copy(z16_ref, cb_v)
    for t in range(RW // RP):
        pltpu.sync_copy(cb_v, cnt_sp.at[pl.ds(sid * RW + t * RP, RP)])
    plsc.subcore_barrier()
    base = wid * EW

    @pl.loop(0, NCHUNK)
    def _chunk(it):
        pltpu.sync_copy(dst_ref.at[pl.ds(base + it * CH, CH)], dst_v)
        pltpu.sync_copy(ones_v, cnt_sp.at[dst_v], add=True)

    plsc.subcore_barrier()
    for t in range(RW // RP):
        r0 = sid * RW + t * RP
        pltpu.sync_copy(cnt_sp.at[pl.ds(r0, RP)], cb_v)
        pltpu.sync_copy(cb_v, cnt_out.at[cid, pl.ds(r0, RP)])


def _cnt_pass(dst):
    fn = pl.kernel(
        _cnt_body,
        out_type=jax.ShapeDtypeStruct((NC, NP, 16), jnp.float32),
        mesh=_get_mesh(),
        scratch_types=[
            pltpu.VMEM((CH,), jnp.int32),
            pltpu.VMEM((RP, 16), jnp.float32),
            pltpu.VMEM((CH, 16), jnp.float32),
            pltpu.VMEM_SHARED((NP, 16), jnp.float32),
            pltpu.SemaphoreType.DMA,
        ],
        compiler_params=pltpu.CompilerParams(use_tc_tiling_on_sc=False),
    )
    return fn(dst, jnp.zeros((RP, 16), jnp.float32),
              jnp.ones((CH, 16), jnp.float32))


def _bn_body(agg_ref, cnt_ref, x_ref, r_ref, g_ref, b_ref, o_ref):
    cnt = cnt_ref[0, :, 0:1] + cnt_ref[1, :, 0:1]
    out = (agg_ref[0] + agg_ref[1]) / jnp.clip(cnt, 1.0, None)
    out = out + jnp.dot(x_ref[...], r_ref[...],
                        preferred_element_type=jnp.float32)
    m = jnp.mean(out, axis=0, keepdims=True)
    v = jnp.mean((out - m) * (out - m), axis=0, keepdims=True)
    y = (out - m) / jnp.sqrt(v + 1e-5) * g_ref[...] + b_ref[...]
    o_ref[...] = jnp.maximum(y, 0.0)


def _bn_layer(agg, cnt, x, r, g, b):
    return pl.pallas_call(
        _bn_body,
        out_shape=jax.ShapeDtypeStruct((N, C), jnp.float32),
    )(agg, cnt, x, r.astype(jnp.float32), g.reshape(1, C), b.reshape(1, C))


def _head_body(aggr_ref, aggco_ref, cnt_ref, x2_ref, x3_ref, rr_ref, rco_ref,
               bias_ref, or_ref, oco_ref):
    cnt = jnp.clip(cnt_ref[0, :, 0:1] + cnt_ref[1, :, 0:1], 1.0, None)
    or_ref[...] = ((aggr_ref[0] + aggr_ref[1]) / cnt
                   + jnp.dot(x2_ref[...], rr_ref[...],
                             preferred_element_type=jnp.float32)
                   + bias_ref[0:1])
    oco_ref[...] = ((aggco_ref[0] + aggco_ref[1]) / cnt
                    + jnp.dot(x3_ref[...], rco_ref[...],
                              preferred_element_type=jnp.float32)
                    + bias_ref[1:2])


def _head_fin(aggr, aggco, cnt, x2, x3, rrp, rcop, bias):
    return pl.pallas_call(
        _head_body,
        out_shape=[jax.ShapeDtypeStruct((N, 16), jnp.float32),
                   jax.ShapeDtypeStruct((N, 16), jnp.float32)],
    )(aggr, aggco, cnt, x2, x3, rrp, rcop, bias)


NPAD = 10240
WCH = 1024


def _win_body(p_ref, o_ref):
    i = pl.program_id(0)

    @pl.when(i == 0)
    def _():
        o_ref[...] = jnp.full((8, NCELL), -1.0, jnp.float32)

    p0 = p_ref[:, 0:1]
    p1 = p_ref[:, 1:2]
    bt = p_ref[:, 2:3]
    xi = jnp.clip(jnp.floor(p0 * (1.0 * GW)), 0.0, GW - 1.0)
    yi = jnp.clip(jnp.floor(p1 * (1.0 * GH)), 0.0, GH - 1.0)
    cell = bt * (GH * GW) + yi * GW + xi
    ids = (i * WCH
           + lax.broadcasted_iota(jnp.int32, (WCH, 1), 0)).astype(jnp.float32)
    valid = ids < (1.0 * N)
    cells = lax.broadcasted_iota(jnp.int32, (1, NCELL), 1).astype(jnp.float32)
    hit = (cell == cells) & valid
    val = jnp.where(hit, ids, -1.0)
    mx = jnp.max(val, axis=0, keepdims=True)
    o_ref[0:1] = jnp.maximum(o_ref[0:1], mx)

    @pl.when(i == (NPAD // WCH) - 1)
    def _():
        o_ref[1:2] = jnp.maximum(o_ref[0:1], 0.0)


def _winner(pos8):
    return pl.pallas_call(
        _win_body,
        grid=(NPAD // WCH,),
        in_specs=[pl.BlockSpec((WCH, 8), lambda i: (i, 0))],
        out_specs=pl.BlockSpec((8, NCELL), lambda i: (0, 0)),
        out_shape=jax.ShapeDtypeStruct((8, NCELL), jnp.float32),
    )(pos8)


def _gath_body(widx_ref, vr_ref, vco_ref, or_ref, oco_ref, idx_v, ra_v, rb_v,
               sem):
    cid = lax.axis_index("c")
    sid = lax.axis_index("s")
    wid = cid * NS + sid
    w0 = wid * CELL_W
    pltpu.sync_copy(widx_ref.at[pl.ds(w0, CELL_W)], idx_v)
    pltpu.async_copy(vr_ref.at[idx_v], ra_v, sem).wait()
    pltpu.async_copy(vco_ref.at[idx_v], rb_v, sem).wait()
    pltpu.sync_copy(ra_v, or_ref.at[pl.ds(w0, CELL_W)])
    pltpu.sync_copy(rb_v, oco_ref.at[pl.ds(w0, CELL_W)])


def _gather_vals(widx, valr, valco):
    fn = pl.kernel(
        _gath_body,
        out_type=(jax.ShapeDtypeStruct((NCELL, 16), jnp.float32),
                  jax.ShapeDtypeStruct((NCELL, 16), jnp.float32)),
        mesh=_get_mesh(),
        compiler_params=pltpu.CompilerParams(use_tc_tiling_on_sc=False),
        scratch_types=[
            pltpu.VMEM((CELL_W,), jnp.int32),
            pltpu.VMEM((CELL_W, 16), jnp.float32),
            pltpu.VMEM((CELL_W, 16), jnp.float32),
            pltpu.SemaphoreType.DMA,
        ],
    )
    return fn(widx, valr, valco)


def _mask_body(grT_ref, gcoT_ref, win_ref, or_ref, oco_ref):
    mask = jnp.where(win_ref[0:1] >= 0.0, 1.0, 0.0)
    or_ref[...] = grT_ref[...] * mask
    oco_ref[...] = gcoT_ref[...] * mask


def _mask_fin(grT, gcoT, win8):
    return pl.pallas_call(
        _mask_body,
        out_shape=[jax.ShapeDtypeStruct((16, NCELL), jnp.float32),
                   jax.ShapeDtypeStruct((16, NCELL), jnp.float32)],
    )(grT, gcoT, win8)


def kernel(x, edge_index, edge_attr, pos, batch, W1, R1, g1, b1, W2, R2, g2,
           b2, W3, R3, g3, b3, Wr, Rr, br, Wc, Rc, bc, Wo, Ro, bo):
    src = edge_index[0].astype(jnp.int32)
    dst = edge_index[1].astype(jnp.int32)

    ea8 = jnp.concatenate([
        edge_attr[:, 0:1], edge_attr[:, 1:2],
        src.astype(jnp.float32)[:, None],
        jnp.zeros((E, 5), jnp.float32)], axis=1).T
    bas8, gi8 = _prep(ea8)
    bas_il = bas8[:4].T.reshape(-1)
    gib_il = gi8[:4].T.reshape(-1)
    gih_il = gi8[4:8].T.reshape(-1)

    wf1 = jnp.transpose(W1, (1, 0, 2)).reshape(C, KT * C)
    wf2 = jnp.transpose(W2, (1, 0, 2)).reshape(C, KT * C)
    wf3 = jnp.transpose(W3, (1, 0, 2)).reshape(C, KT * C)
    wrp = jnp.zeros((KTH, C, 16), jnp.float32).at[:KT, :, :4].set(Wr)
    wrp = jnp.transpose(wrp, (1, 0, 2)).reshape(C, KTH * 16)
    wcop = (jnp.zeros((KTH, C, 16), jnp.float32)
            .at[:KT, :, 0:2].set(Wc).at[:KT, :, 2:3].set(Wo))
    wcop = jnp.transpose(wcop, (1, 0, 2)).reshape(C, KTH * 16)
    rrp = jnp.zeros((C, 16), jnp.float32).at[:, :4].set(Rr)
    rcop = jnp.zeros((C, 16), jnp.float32).at[:, 0:2].set(Rc).at[:, 2:3].set(Ro)
    bias = (jnp.zeros((8, 16), jnp.float32)
            .at[0, :4].set(br).at[1, 0:2].set(bc).at[1, 2].set(bo[0]))

    cnt = _cnt_pass(dst)[:, :N]
    xw1 = _mm(x, wf1).reshape(N * KT, C)
    agg1 = _edge_pass(xw1, gib_il, bas_il, dst, C)[:, :N]
    x1 = _bn_layer(agg1, cnt, x, R1, g1, b1)
    xw2 = _mm(x1, wf2).reshape(N * KT, C)
    agg2 = _edge_pass(xw2, gib_il, bas_il, dst, C)[:, :N]
    x2 = _bn_layer(agg2, cnt, x1, R2, g2, b2)
    xw3 = _mm(x1, wf3).reshape(N * KT, C)
    agg3 = _edge_pass(xw3, gib_il, bas_il, dst, C)[:, :N]
    x3 = _bn_layer(agg3, cnt, x1, R3, g3, b3)
    xwr = _mm(x2, wrp).reshape(N * KTH, 16)
    aggr = _edge_pass(xwr, gih_il, bas_il, dst, 16)[:, :N]
    xwco = _mm(x3, wcop).reshape(N * KTH, 16)
    aggco = _edge_pass(xwco, gih_il, bas_il, dst, 16)[:, :N]
    valr, valco = _head_fin(aggr, aggco, cnt, x2, x3, rrp, rcop, bias)

    pos8 = jnp.concatenate([
        pos, batch.astype(jnp.float32)[:, None],
        jnp.zeros((N, 5), jnp.float32)], axis=1)
    pos8 = jnp.concatenate([pos8, jnp.zeros((NPAD - N, 8), jnp.float32)], 0)
    win8 = _winner(pos8)
    widx = win8[1].astype(jnp.int32)
    gr, gco = _gather_vals(widx, valr, valco)
    grT, gcoT = _mask_fin(gr.T, gco.T, win8)

    reg_out = grT[:4].reshape(4, B, GH, GW).transpose(1, 0, 2, 3)
    cls_out = gcoT[:2].reshape(2, B, GH, GW).transpose(1, 0, 2, 3)
    obj_out = gcoT[2:3].reshape(1, B, GH, GW).transpose(1, 0, 2, 3)
    return (cls_out, reg_out, obj_out)
